# baseline (speedup 1.0000x reference)
import functools

import jax
import jax.numpy as jnp
from jax.experimental import pallas as pl
from jax.experimental.pallas import tpu as pltpu

N = 10000
NPAD = 10240
E = 320000
EPAD = 327680
D_IN = 128
D_HID = 128
N_CLASSES = 40
DC_PAD = 64
EPS = 1e-5


def _norms_body(d_ref, o_ref):
    d = d_ref[...]
    deg_out = d[0, 0] + d[1, 0]
    deg_in = d[0, 1] + d[1, 1]
    ns = jnp.where(deg_out > 0, jax.lax.rsqrt(jnp.maximum(deg_out, 1e-30)), 0.0)
    nd = jnp.where(deg_in > 0, jax.lax.rsqrt(jnp.maximum(deg_in, 1e-30)), 0.0)
    o_ref[...] = jnp.stack([ns, nd])


def _norms_tc(degp):
    return pl.pallas_call(
        _norms_body,
        out_shape=jax.ShapeDtypeStruct((2, NPAD), jnp.float32),
    )(degp)


def _layer_in_body(x_ref, n_ref, w_ref, o_ref):
    ns = n_ref[0, :]
    h = x_ref[...] * ns[:, None]
    o_ref[...] = jnp.dot(h, w_ref[...], preferred_element_type=jnp.float32)


def _layer_in_tc(xpad, norms, W):
    return pl.pallas_call(
        _layer_in_body,
        out_shape=jax.ShapeDtypeStruct((NPAD, W.shape[1]), jnp.float32),
    )(xpad, norms, W)


def _mid_body(p_ref, n_ref, b_ref, g_ref, be_ref, w_ref, o_ref):
    t = p_ref[0, :N, :] + p_ref[1, :N, :]
    nd = n_ref[1, :N]
    t = t * nd[:, None] + b_ref[...]
    mu = jnp.mean(t, axis=0)
    var = jnp.mean((t - mu) ** 2, axis=0)
    y = (t - mu) * jax.lax.rsqrt(var + EPS) * g_ref[...] + be_ref[...]
    y = jnp.maximum(y, 0.0)
    ns = n_ref[0, :N]
    h = y * ns[:, None]
    o_ref[:N, :] = jnp.dot(h, w_ref[...], preferred_element_type=jnp.float32)
    o_ref[N:, :] = jnp.zeros((NPAD - N, o_ref.shape[1]), jnp.float32)


def _layer_mid_tc(parts, norms, b, g, be, W):
    return pl.pallas_call(
        _mid_body,
        out_shape=jax.ShapeDtypeStruct((NPAD, W.shape[1]), jnp.float32),
    )(parts, norms, b.reshape(1, -1), g.reshape(1, -1), be.reshape(1, -1), W)


def _final_body(p_ref, n_ref, b_ref, o_ref):
    t = p_ref[0, :N, :N_CLASSES] + p_ref[1, :N, :N_CLASSES]
    nd = n_ref[1, :N]
    o_ref[...] = t * nd[:, None] + b_ref[...]


def _final_tc(parts, norms, b2):
    return pl.pallas_call(
        _final_body,
        out_shape=jax.ShapeDtypeStruct((N, N_CLASSES), jnp.float32),
    )(parts, norms, b2.reshape(1, -1))


def _deg_sc(srcp, dstp):
    ones = jnp.ones((EPAD,), jnp.float32)
    do = jnp.zeros((NPAD,), jnp.float32).at[srcp].add(ones)
    di = jnp.zeros((NPAD,), jnp.float32).at[dstp].add(ones)
    z = jnp.zeros((2, NPAD), jnp.float32)
    return jnp.stack([jnp.stack([do, di]), z])


def _agg_sc(h, srcp, dstp):
    msg = jnp.take(h, srcp, axis=0)
    agg = jnp.zeros((NPAD, h.shape[1]), jnp.float32).at[dstp].add(msg)
    return jnp.stack([agg, jnp.zeros_like(agg)])


def kernel(x, edge_index, W0, b0, g0, be0, W1, b1, g1, be1, W2, b2):
    pad_idx = (N + jnp.arange(EPAD - E, dtype=jnp.int32) % (NPAD - N))
    srcp = jnp.concatenate([edge_index[0], pad_idx])
    dstp = jnp.concatenate([edge_index[1], pad_idx])

    xpad = jnp.pad(x, ((0, NPAD - N), (0, 0)))
    W2p = jnp.pad(W2, ((0, 0), (0, DC_PAD - N_CLASSES)))

    degp = _deg_sc(srcp, dstp)
    norms = _norms_tc(degp)

    h0 = _layer_in_tc(xpad, norms, W0)
    p0 = _agg_sc(h0, srcp, dstp)
    h1 = _layer_mid_tc(p0, norms, b0, g0, be0, W1)
    p1 = _agg_sc(h1, srcp, dstp)
    h2 = _layer_mid_tc(p1, norms, b1, g1, be1, W2p)
    p2 = _agg_sc(h2, srcp, dstp)
    return _final_tc(p2, norms, b2)


# speedup vs baseline: 12.9110x; 12.9110x over previous
import functools

import jax
import jax.numpy as jnp
from jax import lax
from jax.experimental import pallas as pl
from jax.experimental.pallas import tpu as pltpu
from jax.experimental.pallas import tpu_sc as plsc

N = 10000
NPAD = 10240
E = 320000
EPAD = 327680
D_IN = 128
D_HID = 128
DH = 64
N_CLASSES = 40
EPS = 1e-5

NC = 2
NS = 16
NWORK = NC * NS
CHUNK = 128
CPW = EPAD // (NWORK * CHUNK)
CPT = EPAD // (NS * CHUNK)
ROWS_PT = NPAD // NS
DEGW = 16

_MESH = plsc.VectorSubcoreMesh(core_axis_name="c", subcore_axis_name="s")


def _norms_body(d_ref, o_ref):
    d = d_ref[..., 0]
    deg_out = d[0, 0] + d[1, 0]
    deg_in = d[0, 1] + d[1, 1]
    ns = jnp.where(deg_out > 0, jax.lax.rsqrt(jnp.maximum(deg_out, 1e-30)), 0.0)
    nd = jnp.where(deg_in > 0, jax.lax.rsqrt(jnp.maximum(deg_in, 1e-30)), 0.0)
    o_ref[...] = jnp.stack([ns, nd])


def _norms_tc(degp):
    return pl.pallas_call(
        _norms_body,
        out_shape=jax.ShapeDtypeStruct((2, NPAD), jnp.float32),
    )(degp)


def _layer_in_body(x_ref, n_ref, w_ref, o_ref):
    ns = n_ref[0, :]
    h = x_ref[...] * ns[:, None]
    hw = jnp.dot(h, w_ref[...], preferred_element_type=jnp.float32)
    o_ref[0] = hw[:, :DH]
    o_ref[1] = hw[:, DH:]


def _layer_in_tc(xpad, norms, W):
    return pl.pallas_call(
        _layer_in_body,
        out_shape=jax.ShapeDtypeStruct((2, NPAD, DH), jnp.float32),
    )(xpad, norms, W)


def _mid_body(split_out, p_ref, n_ref, b_ref, g_ref, be_ref, w_ref, o_ref):
    t = jnp.concatenate([p_ref[0, :N, :], p_ref[1, :N, :]], axis=1)
    nd = n_ref[1, :N]
    t = t * nd[:, None] + b_ref[...]
    mu = jnp.mean(t, axis=0)
    var = jnp.mean((t - mu) ** 2, axis=0)
    y = (t - mu) * jax.lax.rsqrt(var + EPS) * g_ref[...] + be_ref[...]
    y = jnp.maximum(y, 0.0)
    ns = n_ref[0, :N]
    h = y * ns[:, None]
    hw = jnp.dot(h, w_ref[...], preferred_element_type=jnp.float32)
    if split_out:
        o_ref[0, :N, :] = hw[:, :DH]
        o_ref[1, :N, :] = hw[:, DH:]
        o_ref[:, N:, :] = jnp.zeros((2, NPAD - N, DH), jnp.float32)
    else:
        o_ref[:N, :] = hw
        o_ref[N:, :] = jnp.zeros((NPAD - N, hw.shape[1]), jnp.float32)


def _layer_mid_tc(parts, norms, b, g, be, W):
    split_out = W.shape[1] == D_HID
    out_shape = (jax.ShapeDtypeStruct((2, NPAD, DH), jnp.float32) if split_out
                 else jax.ShapeDtypeStruct((NPAD, W.shape[1]), jnp.float32))
    return pl.pallas_call(
        functools.partial(_mid_body, split_out),
        out_shape=out_shape,
    )(parts, norms, b.reshape(1, -1), g.reshape(1, -1), be.reshape(1, -1), W)


def _final_body(p_ref, n_ref, b_ref, o_ref):
    t = p_ref[0, :N, :] + p_ref[1, :N, :]
    nd = n_ref[1, :N]
    o_ref[...] = t * nd[:, None] + b_ref[...]


def _final_tc(parts, norms, b2):
    return pl.pallas_call(
        _final_body,
        out_shape=jax.ShapeDtypeStruct((N, N_CLASSES), jnp.float32),
    )(parts, norms, b2.reshape(1, -1))


def _deg_body(src_hbm, dst_hbm, ones_hbm, zeros_hbm, out_hbm,
              sidx, didx, ones_v, deg_sh):
    c = lax.axis_index("c")
    s = lax.axis_index("s")
    wid = c * NS + s
    pltpu.sync_copy(ones_hbm, ones_v)
    pltpu.sync_copy(src_hbm.at[pl.ds(wid * CPW, CPW)], sidx)
    pltpu.sync_copy(dst_hbm.at[pl.ds(wid * CPW, CPW)], didx)
    for d in range(2):
        idx = sidx if d == 0 else didx
        pltpu.sync_copy(zeros_hbm, deg_sh.at[pl.ds(s * ROWS_PT, ROWS_PT)])
        plsc.subcore_barrier()

        def chunk(j, carry):
            pltpu.sync_copy(ones_v, deg_sh.at[idx.at[j]], add=True)
            return carry

        lax.fori_loop(0, CPW, chunk, 0)
        plsc.subcore_barrier()
        pltpu.sync_copy(deg_sh.at[pl.ds(s * ROWS_PT, ROWS_PT)],
                        out_hbm.at[c].at[d].at[pl.ds(s * ROWS_PT, ROWS_PT)])
        plsc.subcore_barrier()


@functools.partial(
    pl.kernel,
    out_type=jax.ShapeDtypeStruct((NC, 2, NPAD, DEGW), jnp.float32),
    mesh=_MESH,
    compiler_params=pltpu.CompilerParams(use_tc_tiling_on_sc=False),
    scratch_types=[
        pltpu.VMEM((CPW, CHUNK), jnp.int32),
        pltpu.VMEM((CPW, CHUNK), jnp.int32),
        pltpu.VMEM((CHUNK, DEGW), jnp.float32),
        pltpu.VMEM_SHARED((NPAD, DEGW), jnp.float32),
    ],
)
def _deg_sc(src2d, dst2d, ones_hbm, zeros_hbm, out, sidx, didx, ones_v, deg_sh):
    _deg_body(src2d, dst2d, ones_hbm, zeros_hbm, out, sidx, didx, ones_v, deg_sh)


def _pipe_chunks(n_chunks, h_tab, sidx, didx, rows0, rows1, agg_sh, sem0, sem1):
    pltpu.async_copy(h_tab.at[sidx.at[0]], rows0, sem0)

    def pipe(i, carry):
        j0 = 2 * i
        j1 = j0 + 1
        pltpu.make_async_copy(h_tab.at[sidx.at[j0]], rows0, sem0).wait()
        pltpu.async_copy(h_tab.at[sidx.at[j1]], rows1, sem1)
        pltpu.sync_copy(rows0, agg_sh.at[didx.at[j0]], add=True)
        j2 = jnp.minimum(j1 + 1, n_chunks - 1)

        @pl.when(j1 + 1 < n_chunks)
        def _():
            pltpu.async_copy(h_tab.at[sidx.at[j2]], rows0, sem0)

        pltpu.make_async_copy(h_tab.at[sidx.at[j1]], rows1, sem1).wait()
        pltpu.sync_copy(rows1, agg_sh.at[didx.at[j1]], add=True)
        return carry

    lax.fori_loop(0, n_chunks // 2, pipe, 0)


def _agg_col_body(h_hbm, src_hbm, dst_hbm, zeros_hbm, out_hbm,
                  sidx, didx, rows0, rows1, agg_sh, sem0, sem1):
    c = lax.axis_index("c")
    s = lax.axis_index("s")
    pltpu.sync_copy(zeros_hbm, agg_sh.at[pl.ds(s * ROWS_PT, ROWS_PT)])
    pltpu.sync_copy(src_hbm.at[pl.ds(s * CPT, CPT)], sidx)
    pltpu.sync_copy(dst_hbm.at[pl.ds(s * CPT, CPT)], didx)
    plsc.subcore_barrier()
    _pipe_chunks(CPT, h_hbm.at[c], sidx, didx, rows0, rows1, agg_sh, sem0, sem1)
    plsc.subcore_barrier()
    pltpu.sync_copy(agg_sh.at[pl.ds(s * ROWS_PT, ROWS_PT)],
                    out_hbm.at[c].at[pl.ds(s * ROWS_PT, ROWS_PT)])


@functools.partial(
    pl.kernel,
    out_type=jax.ShapeDtypeStruct((NC, NPAD, DH), jnp.float32),
    mesh=_MESH,
    compiler_params=pltpu.CompilerParams(use_tc_tiling_on_sc=False),
    scratch_types=[
        pltpu.VMEM((CPT, CHUNK), jnp.int32),
        pltpu.VMEM((CPT, CHUNK), jnp.int32),
        pltpu.VMEM((CHUNK, DH), jnp.float32),
        pltpu.VMEM((CHUNK, DH), jnp.float32),
        pltpu.VMEM_SHARED((NPAD, DH), jnp.float32),
        pltpu.SemaphoreType.DMA,
        pltpu.SemaphoreType.DMA,
    ],
)
def _agg_col_sc(h, src2d, dst2d, zeros_hbm, out, *rest):
    _agg_col_body(h, src2d, dst2d, zeros_hbm, out, *rest)


def _agg_fin_body(h_hbm, src_hbm, dst_hbm, zeros_hbm, out_hbm,
                  sidx, didx, rows0, rows1, agg_sh, sem0, sem1):
    c = lax.axis_index("c")
    s = lax.axis_index("s")
    wid = c * NS + s
    pltpu.sync_copy(zeros_hbm, agg_sh.at[pl.ds(s * ROWS_PT, ROWS_PT)])
    pltpu.sync_copy(src_hbm.at[pl.ds(wid * CPW, CPW)], sidx)
    pltpu.sync_copy(dst_hbm.at[pl.ds(wid * CPW, CPW)], didx)
    plsc.subcore_barrier()
    _pipe_chunks(CPW, h_hbm, sidx, didx, rows0, rows1, agg_sh, sem0, sem1)
    plsc.subcore_barrier()
    pltpu.sync_copy(agg_sh.at[pl.ds(s * ROWS_PT, ROWS_PT)],
                    out_hbm.at[c].at[pl.ds(s * ROWS_PT, ROWS_PT)])


@functools.partial(
    pl.kernel,
    out_type=jax.ShapeDtypeStruct((NC, NPAD, N_CLASSES), jnp.float32),
    mesh=_MESH,
    compiler_params=pltpu.CompilerParams(use_tc_tiling_on_sc=False),
    scratch_types=[
        pltpu.VMEM((CPW, CHUNK), jnp.int32),
        pltpu.VMEM((CPW, CHUNK), jnp.int32),
        pltpu.VMEM((CHUNK, N_CLASSES), jnp.float32),
        pltpu.VMEM((CHUNK, N_CLASSES), jnp.float32),
        pltpu.VMEM_SHARED((NPAD, N_CLASSES), jnp.float32),
        pltpu.SemaphoreType.DMA,
        pltpu.SemaphoreType.DMA,
    ],
)
def _agg_fin_sc(h, src2d, dst2d, zeros_hbm, out, *rest):
    _agg_fin_body(h, src2d, dst2d, zeros_hbm, out, *rest)


def kernel(x, edge_index, W0, b0, g0, be0, W1, b1, g1, be1, W2, b2):
    pad_idx = (N + jnp.arange(EPAD - E, dtype=jnp.int32) % (NPAD - N))
    src2d = jnp.concatenate([edge_index[0], pad_idx]).reshape(EPAD // CHUNK, CHUNK)
    dst2d = jnp.concatenate([edge_index[1], pad_idx]).reshape(EPAD // CHUNK, CHUNK)

    xpad = jnp.pad(x, ((0, NPAD - N), (0, 0)))

    ones_deg = jnp.ones((CHUNK, DEGW), jnp.float32)
    zeros_deg = jnp.zeros((ROWS_PT, DEGW), jnp.float32)
    zeros_dh = jnp.zeros((ROWS_PT, DH), jnp.float32)
    zeros_fin = jnp.zeros((ROWS_PT, N_CLASSES), jnp.float32)

    degp = _deg_sc(src2d, dst2d, ones_deg, zeros_deg)
    norms = _norms_tc(degp)

    h0 = _layer_in_tc(xpad, norms, W0)
    p0 = _agg_col_sc(h0, src2d, dst2d, zeros_dh)
    h1 = _layer_mid_tc(p0, norms, b0, g0, be0, W1)
    p1 = _agg_col_sc(h1, src2d, dst2d, zeros_dh)
    h2 = _layer_mid_tc(p1, norms, b1, g1, be1, W2)
    p2 = _agg_fin_sc(h2, src2d, dst2d, zeros_fin)
    return _final_tc(p2, norms, b2)


# speedup vs baseline: 16.0411x; 1.2424x over previous
import functools

import jax
import jax.numpy as jnp
from jax import lax
from jax.experimental import pallas as pl
from jax.experimental.pallas import tpu as pltpu
from jax.experimental.pallas import tpu_sc as plsc

N = 10000
NPAD = 10240
E = 320000
EPAD = 327680
D_IN = 128
D_HID = 128
DH = 64
N_CLASSES = 40
EPS = 1e-5

NC = 2
NS = 16
NWORK = NC * NS
CHUNK = 128
CPW = EPAD // (NWORK * CHUNK)
CPT = EPAD // (NS * CHUNK)
ROWS_PT = NPAD // NS
DEGW = 16

_MESH = plsc.VectorSubcoreMesh(core_axis_name="c", subcore_axis_name="s")


def _norms_body(d_ref, o_ref):
    d = d_ref[..., 0]
    deg_out = d[0, 0] + d[1, 0]
    deg_in = d[0, 1] + d[1, 1]
    ns = jnp.where(deg_out > 0, jax.lax.rsqrt(jnp.maximum(deg_out, 1e-30)), 0.0)
    nd = jnp.where(deg_in > 0, jax.lax.rsqrt(jnp.maximum(deg_in, 1e-30)), 0.0)
    o_ref[...] = jnp.stack([ns, nd])


def _norms_tc(degp):
    return pl.pallas_call(
        _norms_body,
        out_shape=jax.ShapeDtypeStruct((2, NPAD), jnp.float32),
    )(degp)


def _layer_in_body(x_ref, d_ref, w_ref, o_ref, n_ref):
    _norms_body(d_ref, n_ref)
    ns = n_ref[0, :]
    h = x_ref[...] * ns[:, None]
    hw = jnp.dot(h, w_ref[...], preferred_element_type=jnp.float32)
    o_ref[0] = hw[:, :DH]
    o_ref[1] = hw[:, DH:]


def _layer_in_tc(xpad, degp, W):
    return pl.pallas_call(
        _layer_in_body,
        out_shape=(jax.ShapeDtypeStruct((2, NPAD, DH), jnp.float32),
                   jax.ShapeDtypeStruct((2, NPAD), jnp.float32)),
    )(xpad, degp, W)


def _mid_body(split_out, p_ref, n_ref, b_ref, g_ref, be_ref, w_ref, o_ref):
    t = jnp.concatenate([p_ref[0, :N, :], p_ref[1, :N, :]], axis=1)
    nd = n_ref[1, :N]
    t = t * nd[:, None] + b_ref[...]
    mu = jnp.mean(t, axis=0)
    var = jnp.mean((t - mu) ** 2, axis=0)
    y = (t - mu) * jax.lax.rsqrt(var + EPS) * g_ref[...] + be_ref[...]
    y = jnp.maximum(y, 0.0)
    ns = n_ref[0, :N]
    h = y * ns[:, None]
    hw = jnp.dot(h, w_ref[...], preferred_element_type=jnp.float32)
    if split_out:
        o_ref[0, :N, :] = hw[:, :DH]
        o_ref[1, :N, :] = hw[:, DH:]
        o_ref[:, N:, :] = jnp.zeros((2, NPAD - N, DH), jnp.float32)
    else:
        o_ref[:N, :] = hw
        o_ref[N:, :] = jnp.zeros((NPAD - N, hw.shape[1]), jnp.float32)


def _layer_mid_tc(parts, norms, b, g, be, W):
    split_out = W.shape[1] == D_HID
    out_shape = (jax.ShapeDtypeStruct((2, NPAD, DH), jnp.float32) if split_out
                 else jax.ShapeDtypeStruct((NPAD, W.shape[1]), jnp.float32))
    return pl.pallas_call(
        functools.partial(_mid_body, split_out),
        out_shape=out_shape,
    )(parts, norms, b.reshape(1, -1), g.reshape(1, -1), be.reshape(1, -1), W)


def _final_body(p_ref, n_ref, b_ref, o_ref):
    t = p_ref[0, :N, :] + p_ref[1, :N, :]
    nd = n_ref[1, :N]
    o_ref[...] = t * nd[:, None] + b_ref[...]


def _final_tc(parts, norms, b2):
    return pl.pallas_call(
        _final_body,
        out_shape=jax.ShapeDtypeStruct((N, N_CLASSES), jnp.float32),
    )(parts, norms, b2.reshape(1, -1))


def _deg_body(src_hbm, dst_hbm, ones_hbm, zeros_hbm, out_hbm,
              sidx, didx, ones_v, deg_sh, dsem):
    c = lax.axis_index("c")
    s = lax.axis_index("s")
    wid = c * NS + s
    pltpu.sync_copy(ones_hbm, ones_v)
    pltpu.sync_copy(src_hbm.at[pl.ds(wid * CPW, CPW)], sidx)
    pltpu.sync_copy(dst_hbm.at[pl.ds(wid * CPW, CPW)], didx)
    for d in range(2):
        idx = sidx if d == 0 else didx
        pltpu.sync_copy(zeros_hbm, deg_sh.at[pl.ds(s * ROWS_PT, ROWS_PT)])
        plsc.subcore_barrier()

        def chunk(j, carry):
            pltpu.async_copy(ones_v, deg_sh.at[idx.at[j]], dsem, add=True)
            return carry

        lax.fori_loop(0, CPW, chunk, 0)

        def drain(j, carry):
            pltpu.make_async_copy(ones_v, deg_sh.at[idx.at[0]], dsem).wait()
            return carry

        lax.fori_loop(0, CPW, drain, 0)
        plsc.subcore_barrier()
        pltpu.sync_copy(deg_sh.at[pl.ds(s * ROWS_PT, ROWS_PT)],
                        out_hbm.at[c].at[d].at[pl.ds(s * ROWS_PT, ROWS_PT)])
        plsc.subcore_barrier()


@functools.partial(
    pl.kernel,
    out_type=jax.ShapeDtypeStruct((NC, 2, NPAD, DEGW), jnp.float32),
    mesh=_MESH,
    compiler_params=pltpu.CompilerParams(use_tc_tiling_on_sc=False),
    scratch_types=[
        pltpu.VMEM((CPW, CHUNK), jnp.int32),
        pltpu.VMEM((CPW, CHUNK), jnp.int32),
        pltpu.VMEM((CHUNK, DEGW), jnp.float32),
        pltpu.VMEM_SHARED((NPAD, DEGW), jnp.float32),
        pltpu.SemaphoreType.DMA,
    ],
)
def _deg_sc(src2d, dst2d, ones_hbm, zeros_hbm, out, *rest):
    _deg_body(src2d, dst2d, ones_hbm, zeros_hbm, out, *rest)


NBUF = 4


def _pipe_chunks(n_chunks, h_tab, sidx, didx, bufs, agg_sh, gsems, ssems):
    for b in range(NBUF):
        pltpu.async_copy(h_tab.at[sidx.at[b]], bufs[b], gsems[b])

    def pipe(i, carry):
        for b in range(NBUF):
            j = i * NBUF + b
            pltpu.make_async_copy(h_tab.at[sidx.at[j]], bufs[b], gsems[b]).wait()
            pltpu.async_copy(bufs[b], agg_sh.at[didx.at[j]], ssems[b], add=True)
            jn = jnp.minimum(j + NBUF, n_chunks - 1)

            @pl.when(j + NBUF < n_chunks)
            def _():
                pltpu.make_async_copy(bufs[b], agg_sh.at[didx.at[j]],
                                      ssems[b]).wait()
                pltpu.async_copy(h_tab.at[sidx.at[jn]], bufs[b], gsems[b])
        return carry

    lax.fori_loop(0, n_chunks // NBUF, pipe, 0)
    for b in range(NBUF):
        pltpu.make_async_copy(bufs[b], agg_sh.at[didx.at[0]], ssems[b]).wait()


def _agg_col_body(h_hbm, src_hbm, dst_hbm, zeros_hbm, out_hbm,
                  sidx, didx, b0, b1, b2, b3, agg_sh,
                  g0, g1, g2, g3, s0, s1, s2, s3):
    bufs = (b0, b1, b2, b3)
    gsems = (g0, g1, g2, g3)
    ssems = (s0, s1, s2, s3)
    c = lax.axis_index("c")
    s = lax.axis_index("s")
    pltpu.sync_copy(zeros_hbm, agg_sh.at[pl.ds(s * ROWS_PT, ROWS_PT)])
    pltpu.sync_copy(src_hbm.at[pl.ds(s * CPT, CPT)], sidx)
    pltpu.sync_copy(dst_hbm.at[pl.ds(s * CPT, CPT)], didx)
    plsc.subcore_barrier()
    _pipe_chunks(CPT, h_hbm.at[c], sidx, didx, bufs, agg_sh, gsems, ssems)
    plsc.subcore_barrier()
    pltpu.sync_copy(agg_sh.at[pl.ds(s * ROWS_PT, ROWS_PT)],
                    out_hbm.at[c].at[pl.ds(s * ROWS_PT, ROWS_PT)])


@functools.partial(
    pl.kernel,
    out_type=jax.ShapeDtypeStruct((NC, NPAD, DH), jnp.float32),
    mesh=_MESH,
    compiler_params=pltpu.CompilerParams(use_tc_tiling_on_sc=False),
    scratch_types=[
        pltpu.VMEM((CPT, CHUNK), jnp.int32),
        pltpu.VMEM((CPT, CHUNK), jnp.int32),
        pltpu.VMEM((CHUNK, DH), jnp.float32),
        pltpu.VMEM((CHUNK, DH), jnp.float32),
        pltpu.VMEM((CHUNK, DH), jnp.float32),
        pltpu.VMEM((CHUNK, DH), jnp.float32),
        pltpu.VMEM_SHARED((NPAD, DH), jnp.float32),
    ] + [pltpu.SemaphoreType.DMA] * 8,
)
def _agg_col_sc(h, src2d, dst2d, zeros_hbm, out, *rest):
    _agg_col_body(h, src2d, dst2d, zeros_hbm, out, *rest)


def _agg_fin_body(h_hbm, src_hbm, dst_hbm, zeros_hbm, out_hbm,
                  sidx, didx, b0, b1, b2, b3, agg_sh,
                  g0, g1, g2, g3, s0, s1, s2, s3):
    bufs = (b0, b1, b2, b3)
    gsems = (g0, g1, g2, g3)
    ssems = (s0, s1, s2, s3)
    c = lax.axis_index("c")
    s = lax.axis_index("s")
    wid = c * NS + s
    pltpu.sync_copy(zeros_hbm, agg_sh.at[pl.ds(s * ROWS_PT, ROWS_PT)])
    pltpu.sync_copy(src_hbm.at[pl.ds(wid * CPW, CPW)], sidx)
    pltpu.sync_copy(dst_hbm.at[pl.ds(wid * CPW, CPW)], didx)
    plsc.subcore_barrier()
    _pipe_chunks(CPW, h_hbm, sidx, didx, bufs, agg_sh, gsems, ssems)
    plsc.subcore_barrier()
    pltpu.sync_copy(agg_sh.at[pl.ds(s * ROWS_PT, ROWS_PT)],
                    out_hbm.at[c].at[pl.ds(s * ROWS_PT, ROWS_PT)])


@functools.partial(
    pl.kernel,
    out_type=jax.ShapeDtypeStruct((NC, NPAD, N_CLASSES), jnp.float32),
    mesh=_MESH,
    compiler_params=pltpu.CompilerParams(use_tc_tiling_on_sc=False),
    scratch_types=[
        pltpu.VMEM((CPW, CHUNK), jnp.int32),
        pltpu.VMEM((CPW, CHUNK), jnp.int32),
        pltpu.VMEM((CHUNK, N_CLASSES), jnp.float32),
        pltpu.VMEM((CHUNK, N_CLASSES), jnp.float32),
        pltpu.VMEM((CHUNK, N_CLASSES), jnp.float32),
        pltpu.VMEM((CHUNK, N_CLASSES), jnp.float32),
        pltpu.VMEM_SHARED((NPAD, N_CLASSES), jnp.float32),
    ] + [pltpu.SemaphoreType.DMA] * 8,
)
def _agg_fin_sc(h, src2d, dst2d, zeros_hbm, out, *rest):
    _agg_fin_body(h, src2d, dst2d, zeros_hbm, out, *rest)


def kernel(x, edge_index, W0, b0, g0, be0, W1, b1, g1, be1, W2, b2):
    pad_idx = (N + jnp.arange(EPAD - E, dtype=jnp.int32) % (NPAD - N))
    src2d = jnp.concatenate([edge_index[0], pad_idx]).reshape(EPAD // CHUNK, CHUNK)
    dst2d = jnp.concatenate([edge_index[1], pad_idx]).reshape(EPAD // CHUNK, CHUNK)

    xpad = jnp.pad(x, ((0, NPAD - N), (0, 0)))

    ones_deg = jnp.ones((CHUNK, DEGW), jnp.float32)
    zeros_deg = jnp.zeros((ROWS_PT, DEGW), jnp.float32)
    zeros_dh = jnp.zeros((ROWS_PT, DH), jnp.float32)
    zeros_fin = jnp.zeros((ROWS_PT, N_CLASSES), jnp.float32)

    degp = _deg_sc(src2d, dst2d, ones_deg, zeros_deg)
    h0, norms = _layer_in_tc(xpad, degp, W0)
    p0 = _agg_col_sc(h0, src2d, dst2d, zeros_dh)
    h1 = _layer_mid_tc(p0, norms, b0, g0, be0, W1)
    p1 = _agg_col_sc(h1, src2d, dst2d, zeros_dh)
    h2 = _layer_mid_tc(p1, norms, b1, g1, be1, W2)
    p2 = _agg_fin_sc(h2, src2d, dst2d, zeros_fin)
    return _final_tc(p2, norms, b2)
